# baseline (speedup 1.0000x reference)
import functools

import jax
import jax.numpy as jnp
from jax import lax
from jax.experimental import pallas as pl
from jax.experimental.pallas import tpu as pltpu
from jax.experimental.pallas import tpu_sc as plsc

_B, _S, _M, _D = 1024, 20, 20, 64
_NP_IN = _S * _M
_NP_OUT = _S * 48
_NC, _NS = 2, 16
_NW = _NC * _NS
_NTASK = _S * 6
_TPW = (_NTASK + _NW - 1) // _NW
_H = 32
_HQ = 16


def _bcast_task(xt_hbm, out_hbm, bufs, sems, u):
    s_idx = u // 4
    j4 = u % 4
    base = jnp.where(j4 == 0, 0, jnp.where(j4 == 1, 1,
                     jnp.where(j4 == 2, 10, 11)))
    j = jnp.where(j4 == 0, 0, jnp.where(j4 == 1, 1,
                  jnp.where(j4 == 2, 3, 4)))
    src = s_idx * _M + base
    writes = []
    for h in range(2):
        pltpu.sync_copy(xt_hbm.at[src, pl.ds(h * _H, _H)], bufs[h])
        for k in range(8):
            dst = s_idx * 48 + k * 6 + j
            writes.append(pltpu.async_copy(
                bufs[h], out_hbm.at[dst, pl.ds(h * _H, _H)], sems[h]))
    for wdma in writes:
        wdma.wait()


def _bcast_task_spmem(xt_hbm, out_hbm, spmem, ssems, sid, u):
    s_idx = u // 4
    j4 = u % 4
    base = jnp.where(j4 == 0, 0, jnp.where(j4 == 1, 1,
                     jnp.where(j4 == 2, 10, 11)))
    j = jnp.where(j4 == 0, 0, jnp.where(j4 == 1, 1,
                  jnp.where(j4 == 2, 3, 4)))
    src = s_idx * _M + base
    pend = [None, None]
    for h in range(4):
        p = h % 2
        if pend[p] is not None:
            for wdma in pend[p]:
                wdma.wait()
        pltpu.sync_copy(xt_hbm.at[src, pl.ds(h * _HQ, _HQ)],
                        spmem.at[sid, p])
        pend[p] = [pltpu.async_copy(
            spmem.at[sid, p],
            out_hbm.at[s_idx * 48 + k * 6 + j, pl.ds(h * _HQ, _HQ)],
            ssems[p]) for k in range(8)]
    for grp in pend:
        if grp is not None:
            for wdma in grp:
                wdma.wait()


def _uniq_task(xt_hbm, out_hbm, bufs, sems, v):
    s_idx = v // 4
    q = v % 4
    j = jnp.where(q < 2, 2, 5)
    base = jnp.where(q < 2, 2, 12)
    k0 = jnp.where(q % 2 == 0, 0, 4)
    writes = [None, None, None]
    for kk in range(4):
        k = k0 + kk
        src = s_idx * _M + base + k
        dst = s_idx * 48 + k * 6 + j
        for h in range(2):
            c = 2 * kk + h
            r = c % 3
            if writes[r] is not None:
                writes[r].wait()
            pltpu.sync_copy(xt_hbm.at[src, pl.ds(h * _H, _H)], bufs[r])
            writes[r] = pltpu.async_copy(
                bufs[r], out_hbm.at[dst, pl.ds(h * _H, _H)], sems[r])
    for wdma in writes:
        if wdma is not None:
            wdma.wait()


def _body(xt_hbm, out_hbm, b0, b1, b2, spmem, s0, s1, s2, ss0, ss1):
    sid = lax.axis_index("s")
    wid = sid * _NC + lax.axis_index("c")
    bufs, sems, ssems = (b0, b1, b2), (s0, s1, s2), (ss0, ss1)
    for i in range(5):
        u = wid + _NW * i

        @pl.when(jnp.logical_and(u < 80, u % 4 != 3))
        def _():
            _bcast_task_spmem(xt_hbm, out_hbm, spmem, ssems, sid, u)

        @pl.when(jnp.logical_and(u < 80, u % 4 == 3))
        def _():
            _bcast_task(xt_hbm, out_hbm, bufs, sems, u)

        @pl.when(u >= 80)
        def _():
            _uniq_task(xt_hbm, out_hbm, bufs, sems, u - 80)


@jax.jit
def _recombine(xt):
    mesh = plsc.VectorSubcoreMesh(
        core_axis_name="c", subcore_axis_name="s",
        num_cores=_NC, num_subcores=_NS)
    scratch = [pltpu.VMEM((_H, _B), jnp.float32) for _ in range(3)]
    scratch += [pltpu.VMEM_SHARED((_NS, 2, _HQ, _B), jnp.float32)]
    scratch += [pltpu.SemaphoreType.DMA] * 5
    return pl.kernel(
        _body,
        out_type=jax.ShapeDtypeStruct((_NP_OUT, _D, _B), jnp.float32),
        mesh=mesh,
        scratch_types=scratch,
    )(xt)


def kernel(x):
    b, s, m, d = x.shape
    xt = jnp.transpose(x, (1, 2, 3, 0)).reshape(s * m, d, b)
    out = _recombine(xt)
    out = out.reshape(s, 8, 6, d, b).transpose(4, 0, 1, 2, 3)
    return out


# speedup vs baseline: 1.1159x; 1.1159x over previous
import functools

import jax
import jax.numpy as jnp
from jax import lax
from jax.experimental import pallas as pl
from jax.experimental.pallas import tpu as pltpu
from jax.experimental.pallas import tpu_sc as plsc

_B, _S, _M, _D = 1024, 20, 20, 64
_NP_IN = _S * _M
_NP_OUT = _S * 48
_NC, _NS = 2, 16
_NW = _NC * _NS
_NTASK = _S * 6
_TPW = (_NTASK + _NW - 1) // _NW
_H = 32


def _bcast_task(xt_hbm, out_hbm, bufs, sems, u):
    s_idx = u // 4
    j4 = u % 4
    base = jnp.where(j4 == 0, 0, jnp.where(j4 == 1, 1,
                     jnp.where(j4 == 2, 10, 11)))
    j = jnp.where(j4 == 0, 0, jnp.where(j4 == 1, 1,
                  jnp.where(j4 == 2, 3, 4)))
    src = s_idx * _M + base
    writes = []
    for h in range(2):
        pltpu.sync_copy(xt_hbm.at[src, pl.ds(h * _H, _H)], bufs[h])
        for k in range(8):
            dst = s_idx * 48 + k * 6 + j
            writes.append(pltpu.async_copy(
                bufs[h], out_hbm.at[dst, pl.ds(h * _H, _H)], sems[h]))
    for wdma in writes:
        wdma.wait()


def _uniq_task(xt_hbm, out_hbm, bufs, sems, v):
    s_idx = v // 4
    q = v % 4
    j = jnp.where(q < 2, 2, 5)
    base = jnp.where(q < 2, 2, 12)
    k0 = jnp.where(q % 2 == 0, 0, 4)
    writes = [None, None, None]
    for kk in range(4):
        k = k0 + kk
        src = s_idx * _M + base + k
        dst = s_idx * 48 + k * 6 + j
        for h in range(2):
            c = 2 * kk + h
            r = c % 3
            if writes[r] is not None:
                writes[r].wait()
            pltpu.sync_copy(xt_hbm.at[src, pl.ds(h * _H, _H)], bufs[r])
            writes[r] = pltpu.async_copy(
                bufs[r], out_hbm.at[dst, pl.ds(h * _H, _H)], sems[r])
    for wdma in writes:
        if wdma is not None:
            wdma.wait()


def _body(xt_hbm, out_hbm, b0, b1, b2, s0, s1, s2):
    wid = lax.axis_index("s") * _NC + lax.axis_index("c")
    bufs, sems = (b0, b1, b2), (s0, s1, s2)
    for i in range(5):
        u = wid + _NW * i

        @pl.when(u < 80)
        def _():
            _bcast_task(xt_hbm, out_hbm, bufs, sems, u)

        @pl.when(u >= 80)
        def _():
            _uniq_task(xt_hbm, out_hbm, bufs, sems, u - 80)


@jax.jit
def _recombine(xt):
    mesh = plsc.VectorSubcoreMesh(
        core_axis_name="c", subcore_axis_name="s",
        num_cores=_NC, num_subcores=_NS)
    scratch = [pltpu.VMEM((_H, _B), jnp.float32) for _ in range(3)]
    scratch += [pltpu.SemaphoreType.DMA] * 3
    return pl.kernel(
        _body,
        out_type=jax.ShapeDtypeStruct((_NP_OUT, _D, _B), jnp.float32),
        mesh=mesh,
        scratch_types=scratch,
    )(xt)


def kernel(x):
    b, s, m, d = x.shape
    xt = jnp.transpose(x, (1, 2, 3, 0)).reshape(s * m, d, b)
    out = _recombine(xt)
    out = out.reshape(s, 8, 6, d, b).transpose(4, 0, 1, 2, 3)
    return out
